# baseline (speedup 1.0000x reference)
import jax
import jax.numpy as jnp
from jax import lax
from jax.experimental import pallas as pl
from jax.experimental.pallas import tpu as pltpu

N = 2048
TOP_K = 64
OBS_RADIUS = 1.0
R = 128
CJ = 128
GRID = N // R
NCHUNKS = N // CJ
MAX_FINITE = 0x7F800000


def _body(states_ref, goals_ref, Wc1_ref, bc1_ref, Wc2_ref, bc2_ref,
          Wd1_ref, bd1_ref, Wd2_ref, bd2_ref, Wd3_ref, bd3_ref,
          Wd4_ref, bd4_ref, out_ref, a_ref, d_ref):
    pid = pl.program_id(0)
    row0 = pid * R
    s_all = states_ref[...]
    s_blk = states_ref[pl.ds(row0, R), :]

    px_a = s_all[:, 0:1]
    py_a = s_all[:, 1:2]
    nrm_a = px_a * px_a + py_a * py_a
    one_a = jnp.ones_like(px_a)
    feat_b = jnp.concatenate([one_a, px_a, py_a, nrm_a], axis=1)
    px_r = s_blk[:, 0:1]
    py_r = s_blk[:, 1:2]
    nrm_r = px_r * px_r + py_r * py_r
    one_r = jnp.ones_like(px_r)
    feat_a = jnp.concatenate([nrm_r, -2.0 * px_r, -2.0 * py_r, one_r], axis=1)
    d2 = lax.dot_general(feat_a, feat_b, (((1,), (1,)), ((), ())),
                         preferred_element_type=jnp.float32)
    d2 = jnp.maximum(d2, 0.0)
    d_ref[...] = d2

    dint = lax.bitcast_convert_type(d2, jnp.int32)

    def bs_body(_, lohi):
        lo, hi = lohi
        mid = lo + ((hi - lo) >> 1)
        cnt = jnp.sum((dint <= mid).astype(jnp.int32), axis=1, keepdims=True)
        ge = cnt >= TOP_K
        return (jnp.where(ge, lo, mid + 1), jnp.where(ge, mid, hi))

    lo0 = jnp.zeros((R, 1), jnp.int32)
    hi0 = jnp.full((R, 1), MAX_FINITE, jnp.int32)
    _, tsel = lax.fori_loop(0, 31, bs_body, (lo0, hi0))

    wc1p = Wc1_ref[0:4, :]
    wc1e = Wc1_ref[4:5, :].reshape(1, 1, 64)
    bc1 = bc1_ref[...].reshape(1, 1, 64)
    wc2 = Wc2_ref[...]
    bc2 = bc2_ref[...]
    a_ref[...] = jnp.dot(s_all, wc1p, preferred_element_type=jnp.float32)
    a_blk = a_ref[pl.ds(row0, R), :]

    def chunk_body(c, acc):
        j0 = c * CJ
        a_ch = a_ref[pl.ds(j0, CJ), :]
        pre = a_blk[:, None, :] - a_ch[None, :, :] + bc1
        ii = row0 + lax.broadcasted_iota(jnp.int32, (R, CJ), 0)
        jj = j0 + lax.broadcasted_iota(jnp.int32, (R, CJ), 1)
        iseye = (ii == jj).astype(jnp.float32)
        pre = pre + iseye[:, :, None] * wc1e
        h1 = jnp.maximum(pre, 0.0).reshape(R * CJ, 64)
        h2 = jnp.maximum(
            jnp.dot(h1, wc2, preferred_element_type=jnp.float32) + bc2, 0.0)
        h2 = h2.reshape(R, CJ, 128)
        dch = d_ref[:, pl.ds(j0, CJ)]
        dich = lax.bitcast_convert_type(dch, jnp.int32)
        keep = ((dich <= tsel) & (jnp.sqrt(dch) < OBS_RADIUS)
                ).astype(jnp.float32)
        h2m = h2 * keep[:, :, None]
        return jnp.maximum(acc, jnp.max(h2m, axis=1))

    acc0 = jnp.zeros((R, 128), jnp.float32)
    x_local = lax.fori_loop(0, NCHUNKS, chunk_body, acc0)

    g_blk = goals_ref[pl.ds(row0, R), :]
    e = jnp.concatenate([s_blk[:, 0:2] - g_blk, s_blk[:, 2:4]], axis=1)
    d1 = jnp.maximum(
        jnp.dot(x_local, Wd1_ref[0:128, :], preferred_element_type=jnp.float32)
        + jnp.dot(e, Wd1_ref[128:132, :], preferred_element_type=jnp.float32)
        + bd1_ref[...], 0.0)
    d2_ = jnp.maximum(
        jnp.dot(d1, Wd2_ref[...], preferred_element_type=jnp.float32)
        + bd2_ref[...], 0.0)
    d3 = jnp.maximum(
        jnp.dot(d2_, Wd3_ref[...], preferred_element_type=jnp.float32)
        + bd3_ref[...], 0.0)
    z = jnp.dot(d3, Wd4_ref[...], preferred_element_type=jnp.float32) \
        + bd4_ref[...]
    kk = 2.0 / (1.0 + jnp.exp(-z)) + 0.2
    ax = -(kk[:, 0:1] * e[:, 0:1] + kk[:, 1:2] * e[:, 2:3])
    ay = -(kk[:, 2:3] * e[:, 1:2] + kk[:, 3:4] * e[:, 3:4])
    out_ref[...] = jnp.concatenate([ax, ay], axis=1)


def kernel(states, goals, Wc1, bc1, Wc2, bc2, Wd1, bd1, Wd2, bd2,
           Wd3, bd3, Wd4, bd4):
    full = lambda shape: pl.BlockSpec(shape, lambda i: tuple(0 for _ in shape))
    in_specs = [
        full((N, 4)), full((N, 2)),
        full((5, 64)), full((1, 64)), full((64, 128)), full((1, 128)),
        full((132, 64)), full((1, 64)), full((64, 128)), full((1, 128)),
        full((128, 64)), full((1, 64)), full((64, 4)), full((1, 4)),
    ]
    out = pl.pallas_call(
        _body,
        grid=(GRID,),
        in_specs=in_specs,
        out_specs=pl.BlockSpec((R, 2), lambda i: (i, 0)),
        out_shape=jax.ShapeDtypeStruct((N, 2), jnp.float32),
        scratch_shapes=[
            pltpu.VMEM((N, 64), jnp.float32),
            pltpu.VMEM((R, N), jnp.float32),
        ],
        compiler_params=pltpu.CompilerParams(
            dimension_semantics=("arbitrary",),
        ),
    )(states, goals,
      Wc1, bc1.reshape(1, 64), Wc2, bc2.reshape(1, 128),
      Wd1, bd1.reshape(1, 64), Wd2, bd2.reshape(1, 128),
      Wd3, bd3.reshape(1, 64), Wd4, bd4.reshape(1, 4))
    return out


# speedup vs baseline: 2.2324x; 2.2324x over previous
import functools
import jax
import jax.numpy as jnp
from jax import lax
from jax.experimental import pallas as pl
from jax.experimental.pallas import tpu as pltpu
from jax.experimental.pallas import tpu_sc as plsc

N = 2048
TOP_K = 64
OBS_RADIUS = 1.0
MAX_FINITE = 0x7F800000

R1B = 128
G1 = N // R1B
R3B = 128
G3 = N // R3B

B = N * TOP_K
NW = 32
BW = B // NW
CH = 128
NCH = BW // CH


def _lane_cumsum_excl(x):
    inc = x
    n = x.shape[-1]
    s = 1
    while s < n:
        z = jnp.zeros(x.shape[:-1] + (s,), x.dtype)
        inc = inc + jnp.concatenate([z, inc[..., :-s]], axis=-1)
        s *= 2
    return inc - x


def _idx_body(states_ref, idx_ref):
    pid = pl.program_id(0)
    row0 = pid * R1B
    s_all = states_ref[...]
    s_blk = states_ref[pl.ds(row0, R1B), :]

    px_a = s_all[:, 0:1]
    py_a = s_all[:, 1:2]
    nrm_a = px_a * px_a + py_a * py_a
    one_a = jnp.ones_like(px_a)
    feat_b = jnp.concatenate([one_a, px_a, py_a, nrm_a], axis=1)
    px_r = s_blk[:, 0:1]
    py_r = s_blk[:, 1:2]
    nrm_r = px_r * px_r + py_r * py_r
    one_r = jnp.ones_like(px_r)
    feat_a = jnp.concatenate([nrm_r, -2.0 * px_r, -2.0 * py_r, one_r], axis=1)
    d2 = lax.dot_general(feat_a, feat_b, (((1,), (1,)), ((), ())),
                         preferred_element_type=jnp.float32)
    d2 = jnp.maximum(d2, 0.0)
    dint = lax.bitcast_convert_type(d2, jnp.int32)

    def bs_body(_, lohi):
        lo, hi = lohi
        mid = lo + ((hi - lo) >> 1)
        cnt = jnp.sum((dint <= mid).astype(jnp.int32), axis=1, keepdims=True)
        ge = cnt >= TOP_K
        return (jnp.where(ge, lo, mid + 1), jnp.where(ge, mid, hi))

    lo0 = jnp.zeros((R1B, 1), jnp.int32)
    hi0 = jnp.full((R1B, 1), MAX_FINITE, jnp.int32)
    _, tsel = lax.fori_loop(0, 31, bs_body, (lo0, hi0))

    m_lt = (dint < tsel).astype(jnp.int32)
    m_eq = (dint == tsel).astype(jnp.int32)
    c_lt = jnp.sum(m_lt, axis=1, keepdims=True)
    p_lt = _lane_cumsum_excl(m_lt)
    p_eq = c_lt + _lane_cumsum_excl(m_eq)
    pos = jnp.where(m_lt == 1, p_lt, p_eq)
    m_all = (m_lt | m_eq) == 1
    jj = lax.broadcasted_iota(jnp.int32, (R1B, N), 1)
    cols = []
    for k in range(TOP_K):
        hit = m_all & (pos == k)
        cols.append(jnp.sum(jnp.where(hit, jj, 0), axis=1, keepdims=True))
    idx_ref[...] = jnp.concatenate(cols, axis=1)


def _make_idx(states):
    return pl.pallas_call(
        _idx_body,
        grid=(G1,),
        in_specs=[pl.BlockSpec((N, 4), lambda i: (0, 0))],
        out_specs=pl.BlockSpec((R1B, TOP_K), lambda i: (i, 0)),
        out_shape=jax.ShapeDtypeStruct((N, TOP_K), jnp.int32),
        compiler_params=pltpu.CompilerParams(
            dimension_semantics=("arbitrary",),
        ),
    )(states)


@functools.cache
def _sc_gather_fn():
    mesh = plsc.VectorSubcoreMesh(core_axis_name="c", subcore_axis_name="s")

    @functools.partial(
        pl.kernel,
        mesh=mesh,
        out_type=jax.ShapeDtypeStruct((B, 128), jnp.float32),
        scratch_types=[
            pltpu.VMEM((BW,), jnp.int32),
            pltpu.VMEM((CH, 128), jnp.float32),
            pltpu.SemaphoreType.DMA,
        ],
    )
    def _sc_gather(table_hbm, idx_hbm, out_hbm, idx_v, rows_v, sem):
        wid = lax.axis_index("s") * 2 + lax.axis_index("c")
        base = wid * BW
        pltpu.sync_copy(idx_hbm.at[pl.ds(base, BW)], idx_v)

        def chunk(ci, carry):
            off = ci * CH
            pltpu.async_copy(
                table_hbm.at[idx_v.at[pl.ds(off, CH)]], rows_v, sem).wait()
            pltpu.sync_copy(rows_v, out_hbm.at[pl.ds(base + off, CH)])
            return carry

        lax.fori_loop(0, NCH, chunk, 0)

    return _sc_gather


def _dense_body(grows_ref, states_ref, goals_ref, Wc1_ref, bc1_ref, Wc2_ref,
                bc2_ref, Wd1_ref, bd1_ref, Wd2_ref, bd2_ref, Wd3_ref, bd3_ref,
                Wd4_ref, bd4_ref, out_ref):
    pid = pl.program_id(0)
    row0 = pid * R3B
    g = grows_ref[...]
    s_blk = states_ref[pl.ds(row0, R3B), :]
    s_rep = lax.broadcast_in_dim(
        s_blk, (R3B, TOP_K, 4), (0, 2)).reshape(R3B * TOP_K, 4)
    diff = s_rep - g[:, 0:4]
    dx = diff[:, 0:1]
    dy = diff[:, 1:2]
    dist = jnp.sqrt(dx * dx + dy * dy)
    az = (jnp.abs(diff[:, 0:1]) + jnp.abs(diff[:, 1:2])
          + jnp.abs(diff[:, 2:3]) + jnp.abs(diff[:, 3:4]))
    eye = (az == 0.0).astype(jnp.float32)
    pre = (jnp.dot(diff, Wc1_ref[0:4, :], preferred_element_type=jnp.float32)
           + eye * Wc1_ref[4:5, :] + bc1_ref[...])
    h1 = jnp.maximum(pre, 0.0)
    h2 = jnp.maximum(
        jnp.dot(h1, Wc2_ref[...], preferred_element_type=jnp.float32)
        + bc2_ref[...], 0.0)
    maskf = (dist < OBS_RADIUS).astype(jnp.float32)
    h2m = h2 * maskf
    x_local = jnp.max(h2m.reshape(R3B, TOP_K, 128), axis=1)

    g_blk = goals_ref[pl.ds(row0, R3B), :]
    e = jnp.concatenate([s_blk[:, 0:2] - g_blk, s_blk[:, 2:4]], axis=1)
    d1 = jnp.maximum(
        jnp.dot(x_local, Wd1_ref[0:128, :], preferred_element_type=jnp.float32)
        + jnp.dot(e, Wd1_ref[128:132, :], preferred_element_type=jnp.float32)
        + bd1_ref[...], 0.0)
    d2_ = jnp.maximum(
        jnp.dot(d1, Wd2_ref[...], preferred_element_type=jnp.float32)
        + bd2_ref[...], 0.0)
    d3 = jnp.maximum(
        jnp.dot(d2_, Wd3_ref[...], preferred_element_type=jnp.float32)
        + bd3_ref[...], 0.0)
    z = jnp.dot(d3, Wd4_ref[...], preferred_element_type=jnp.float32) \
        + bd4_ref[...]
    kk = 2.0 / (1.0 + jnp.exp(-z)) + 0.2
    ax = -(kk[:, 0:1] * e[:, 0:1] + kk[:, 1:2] * e[:, 2:3])
    ay = -(kk[:, 2:3] * e[:, 1:2] + kk[:, 3:4] * e[:, 3:4])
    out_ref[...] = jnp.concatenate([ax, ay], axis=1)


def kernel(states, goals, Wc1, bc1, Wc2, bc2, Wd1, bd1, Wd2, bd2,
           Wd3, bd3, Wd4, bd4):
    idx = _make_idx(states)
    states_pad = jnp.pad(states, ((0, 0), (0, 124)))
    grows = _sc_gather_fn()(states_pad, idx.reshape(B))

    full = lambda shape: pl.BlockSpec(shape, lambda i: tuple(0 for _ in shape))
    in_specs = [
        pl.BlockSpec((R3B * TOP_K, 128), lambda i: (i, 0)),
        full((N, 4)), full((N, 2)),
        full((5, 64)), full((1, 64)), full((64, 128)), full((1, 128)),
        full((132, 64)), full((1, 64)), full((64, 128)), full((1, 128)),
        full((128, 64)), full((1, 64)), full((64, 4)), full((1, 4)),
    ]
    out = pl.pallas_call(
        _dense_body,
        grid=(G3,),
        in_specs=in_specs,
        out_specs=pl.BlockSpec((R3B, 2), lambda i: (i, 0)),
        out_shape=jax.ShapeDtypeStruct((N, 2), jnp.float32),
        compiler_params=pltpu.CompilerParams(
            dimension_semantics=("arbitrary",),
        ),
    )(grows, states, goals,
      Wc1, bc1.reshape(1, 64), Wc2, bc2.reshape(1, 128),
      Wd1, bd1.reshape(1, 64), Wd2, bd2.reshape(1, 128),
      Wd3, bd3.reshape(1, 64), Wd4, bd4.reshape(1, 4))
    return out
